# baseline (speedup 1.0000x reference)
import jax
import jax.numpy as jnp
from jax.experimental import pallas as pl
from jax.experimental.pallas import tpu as pltpu

_K = 16
_DIN = 2048
_DH = 2048
_N = 4096
_T = 256
_HB = 512
_J = _DH // _HB
_NBUF = 4
_S = _K * _J


def _issue_w1(w1_hbm, bufs, sems, s):
    e = s // _J
    j = s % _J
    pltpu.make_async_copy(
        w1_hbm.at[e, :, pl.ds(j * _HB, _HB)],
        bufs.at[s % _NBUF],
        sems.at[s % _NBUF],
    ).start()


def _mlp_kernel(offs_ref, xs_ref, w1_hbm, b1_ref, w2_ref, b2_ref, out_ref,
                bufs, sems):
    e = pl.program_id(0)
    j = pl.program_id(1)
    s = e * _J + j
    start = offs_ref[e]
    end = offs_ref[e + 1]
    start8 = (start // 8) * 8
    nch = jnp.where(end > start, (end - start8 + _T - 1) // _T, 0)

    @pl.when(s == 0)
    def _prologue():
        for i in range(_NBUF):
            _issue_w1(w1_hbm, bufs, sems, i)

    pltpu.make_async_copy(
        w1_hbm.at[e, :, pl.ds(j * _HB, _HB)],
        bufs.at[s % _NBUF],
        sems.at[s % _NBUF],
    ).wait()

    w1b = bufs[s % _NBUF].astype(jnp.bfloat16)
    w2b = w2_ref[0]
    b1b = b1_ref[0, 0]
    b2b = b2_ref[0, 0]

    def body(c, _):
        base = jnp.minimum(start8 + c * _T, _N - _T)
        xb = xs_ref[pl.ds(base, _T), :]
        h = jnp.dot(xb, w1b, preferred_element_type=jnp.float32)
        h = jnp.maximum(h + b1b[None, :], 0.0)
        o = jnp.dot(h.astype(jnp.bfloat16), w2b.astype(jnp.bfloat16),
                    preferred_element_type=jnp.float32)
        rows = base + jax.lax.broadcasted_iota(jnp.int32, (_T, 1), 0)
        mask = (rows >= start) & (rows < end)
        prev = out_ref[pl.ds(base, _T), :]
        acc = jnp.where(j == 0, o + b2b[None, :], prev + o)
        m = jnp.max(acc, axis=-1, keepdims=True)
        p = jnp.exp(acc - m)
        sm = p / jnp.sum(p, axis=-1, keepdims=True)
        val = jnp.where(j == _J - 1, sm, acc)
        out_ref[pl.ds(base, _T), :] = jnp.where(mask, val, prev)
        return 0

    jax.lax.fori_loop(0, nch, body, 0)

    @pl.when(s + _NBUF < _S)
    def _refill():
        _issue_w1(w1_hbm, bufs, sems, s + _NBUF)


def _grouped_mlp(offs, xs, W1, b1, W2, b2, interpret=False):
    return pl.pallas_call(
        _mlp_kernel,
        grid_spec=pltpu.PrefetchScalarGridSpec(
            num_scalar_prefetch=1,
            grid=(_K, _J),
            in_specs=[
                pl.BlockSpec((_N, _DIN), lambda e, j, offs: (0, 0)),
                pl.BlockSpec(memory_space=pltpu.MemorySpace.HBM),
                pl.BlockSpec((1, 1, _HB), lambda e, j, offs: (e, 0, j)),
                pl.BlockSpec((1, _HB, 2), lambda e, j, offs: (e, j, 0)),
                pl.BlockSpec((1, 1, 2), lambda e, j, offs: (e, 0, 0)),
            ],
            out_specs=pl.BlockSpec((_N, 2), lambda e, j, offs: (0, 0)),
            scratch_shapes=[
                pltpu.VMEM((_NBUF, _DIN, _HB), jnp.float32),
                pltpu.SemaphoreType.DMA((_NBUF,)),
            ],
        ),
        out_shape=jax.ShapeDtypeStruct((_N, 2), jnp.float32),
        interpret=interpret,
    )(offs, xs, W1, b1, W2, b2)


def kernel(x, z, W1, b1, W2, b2):
    sort_idx = jnp.argsort(z)
    counts = jnp.bincount(z, length=_K)
    offs = jnp.concatenate(
        [jnp.zeros((1,), jnp.int32), jnp.cumsum(counts).astype(jnp.int32)])
    xs = x[sort_idx].astype(jnp.bfloat16)
    out_sorted = _grouped_mlp(offs, xs, W1, b1[:, None, :], W2, b2[:, None, :])
    return jnp.zeros((_N, 2), jnp.float32).at[sort_idx].set(out_sorted)


# speedup vs baseline: 1.0393x; 1.0393x over previous
import jax
import jax.numpy as jnp
from jax.experimental import pallas as pl
from jax.experimental.pallas import tpu as pltpu

_K = 16
_DIN = 2048
_DH = 2048
_N = 4096
_T = 256
_HB = 512
_J = _DH // _HB
_NBUF = 4
_S = _K * _J


def _issue_w1(w1_hbm, bufs, sems, s):
    e = s // _J
    j = s % _J
    pltpu.make_async_copy(
        w1_hbm.at[e, :, pl.ds(j * _HB, _HB)],
        bufs.at[s % _NBUF],
        sems.at[s % _NBUF],
    ).start()


def _mlp_kernel(offs_ref, xs_ref, w1_hbm, b1_ref, w2_ref, b2_ref, out_ref,
                bufs, sems):
    e = pl.program_id(0)
    j = pl.program_id(1)
    s = e * _J + j
    start = offs_ref[e]
    end = offs_ref[e + 1]
    start8 = (start // 8) * 8
    nch = jnp.where(end > start, (end - start8 + _T - 1) // _T, 0)

    @pl.when(s == 0)
    def _prologue():
        for i in range(_NBUF - 1):
            _issue_w1(w1_hbm, bufs, sems, i)

    @pl.when(s + _NBUF - 1 < _S)
    def _refill():
        _issue_w1(w1_hbm, bufs, sems, s + _NBUF - 1)

    pltpu.make_async_copy(
        w1_hbm.at[e, :, pl.ds(j * _HB, _HB)],
        bufs.at[s % _NBUF],
        sems.at[s % _NBUF],
    ).wait()

    w1b = bufs[s % _NBUF].astype(jnp.bfloat16)
    w2b = w2_ref[0]
    b1b = b1_ref[0, 0]
    b2b = b2_ref[0, 0]

    def body(c, _):
        base = jnp.minimum(start8 + c * _T, _N - _T)
        xb = xs_ref[pl.ds(base, _T), :]
        h = jnp.dot(xb, w1b, preferred_element_type=jnp.float32)
        h = jnp.maximum(h + b1b[None, :], 0.0)
        o = jnp.dot(h.astype(jnp.bfloat16), w2b.astype(jnp.bfloat16),
                    preferred_element_type=jnp.float32)
        rows = base + jax.lax.broadcasted_iota(jnp.int32, (_T, 1), 0)
        mask = (rows >= start) & (rows < end)
        prev = out_ref[pl.ds(base, _T), :]
        acc = jnp.where(j == 0, o + b2b[None, :], prev + o)
        m = jnp.max(acc, axis=-1, keepdims=True)
        p = jnp.exp(acc - m)
        sm = p / jnp.sum(p, axis=-1, keepdims=True)
        val = jnp.where(j == _J - 1, sm, acc)
        out_ref[pl.ds(base, _T), :] = jnp.where(mask, val, prev)
        return 0

    jax.lax.fori_loop(0, nch, body, 0)


def _grouped_mlp(offs, xs, W1, b1, W2, b2, interpret=False):
    return pl.pallas_call(
        _mlp_kernel,
        grid_spec=pltpu.PrefetchScalarGridSpec(
            num_scalar_prefetch=1,
            grid=(_K, _J),
            in_specs=[
                pl.BlockSpec((_N, _DIN), lambda e, j, offs: (0, 0)),
                pl.BlockSpec(memory_space=pltpu.MemorySpace.HBM),
                pl.BlockSpec((1, 1, _HB), lambda e, j, offs: (e, 0, j)),
                pl.BlockSpec((1, _HB, 2), lambda e, j, offs: (e, j, 0)),
                pl.BlockSpec((1, 1, 2), lambda e, j, offs: (e, 0, 0)),
            ],
            out_specs=pl.BlockSpec((_N, 2), lambda e, j, offs: (0, 0)),
            scratch_shapes=[
                pltpu.VMEM((_NBUF, _DIN, _HB), jnp.float32),
                pltpu.SemaphoreType.DMA((_NBUF,)),
            ],
        ),
        out_shape=jax.ShapeDtypeStruct((_N, 2), jnp.float32),
        interpret=interpret,
    )(offs, xs, W1, b1, W2, b2)


def kernel(x, z, W1, b1, W2, b2):
    sort_idx = jnp.argsort(z)
    counts = jnp.bincount(z, length=_K)
    offs = jnp.concatenate(
        [jnp.zeros((1,), jnp.int32), jnp.cumsum(counts).astype(jnp.int32)])
    xs = x[sort_idx].astype(jnp.bfloat16)
    out_sorted = _grouped_mlp(offs, xs, W1, b1[:, None, :], W2, b2[:, None, :])
    return jnp.zeros((_N, 2), jnp.float32).at[sort_idx].set(out_sorted)


# speedup vs baseline: 1.1759x; 1.1315x over previous
import jax
import jax.numpy as jnp
from jax.experimental import pallas as pl
from jax.experimental.pallas import tpu as pltpu

_K = 16
_DIN = 2048
_DH = 2048
_N = 4096
_T = 256
_HB = 1024
_J = _DH // _HB
_NBUF = 3
_S = _K * _J


def _issue_w1(w1_hbm, bufs, sems, s):
    e = s // _J
    j = s % _J
    pltpu.make_async_copy(
        w1_hbm.at[e, :, pl.ds(j * _HB, _HB)],
        bufs.at[s % _NBUF],
        sems.at[s % _NBUF],
    ).start()


def _mlp_kernel(offs_ref, xs_ref, w1_hbm, b1_ref, w2_ref, b2_ref, out_ref,
                bufs, sems):
    e = pl.program_id(0)
    j = pl.program_id(1)
    s = e * _J + j
    start = offs_ref[e]
    end = offs_ref[e + 1]
    start8 = (start // 8) * 8
    nch = jnp.where(end > start, (end - start8 + _T - 1) // _T, 0)

    @pl.when(s == 0)
    def _prologue():
        for i in range(_NBUF - 1):
            _issue_w1(w1_hbm, bufs, sems, i)

    @pl.when(s + _NBUF - 1 < _S)
    def _refill():
        _issue_w1(w1_hbm, bufs, sems, s + _NBUF - 1)

    pltpu.make_async_copy(
        w1_hbm.at[e, :, pl.ds(j * _HB, _HB)],
        bufs.at[s % _NBUF],
        sems.at[s % _NBUF],
    ).wait()

    w1b = bufs[s % _NBUF].astype(jnp.bfloat16)
    w2b = w2_ref[0]
    b1b = b1_ref[0, 0]
    b2b = b2_ref[0, 0]

    def body(c, _):
        base = jnp.minimum(start8 + c * _T, _N - _T)
        xb = xs_ref[pl.ds(base, _T), :]
        h = jnp.dot(xb, w1b, preferred_element_type=jnp.float32)
        h = jnp.maximum(h + b1b[None, :], 0.0)
        o = jnp.dot(h.astype(jnp.bfloat16), w2b.astype(jnp.bfloat16),
                    preferred_element_type=jnp.float32)
        rows = base + jax.lax.broadcasted_iota(jnp.int32, (_T, 1), 0)
        mask = (rows >= start) & (rows < end)
        prev = out_ref[pl.ds(base, _T), :]
        acc = jnp.where(j == 0, o + b2b[None, :], prev + o)
        m = jnp.max(acc, axis=-1, keepdims=True)
        p = jnp.exp(acc - m)
        sm = p / jnp.sum(p, axis=-1, keepdims=True)
        val = jnp.where(j == _J - 1, sm, acc)
        out_ref[pl.ds(base, _T), :] = jnp.where(mask, val, prev)
        return 0

    jax.lax.fori_loop(0, nch, body, 0)


def _grouped_mlp(offs, xs, W1, b1, W2, b2, interpret=False):
    return pl.pallas_call(
        _mlp_kernel,
        grid_spec=pltpu.PrefetchScalarGridSpec(
            num_scalar_prefetch=1,
            grid=(_K, _J),
            in_specs=[
                pl.BlockSpec((_N, _DIN), lambda e, j, offs: (0, 0)),
                pl.BlockSpec(memory_space=pltpu.MemorySpace.HBM),
                pl.BlockSpec((1, 1, _HB), lambda e, j, offs: (e, 0, j)),
                pl.BlockSpec((1, _HB, 2), lambda e, j, offs: (e, j, 0)),
                pl.BlockSpec((1, 1, 2), lambda e, j, offs: (e, 0, 0)),
            ],
            out_specs=pl.BlockSpec((_N, 2), lambda e, j, offs: (0, 0)),
            scratch_shapes=[
                pltpu.VMEM((_NBUF, _DIN, _HB), jnp.float32),
                pltpu.SemaphoreType.DMA((_NBUF,)),
            ],
        ),
        out_shape=jax.ShapeDtypeStruct((_N, 2), jnp.float32),
        interpret=interpret,
    )(offs, xs, W1, b1, W2, b2)


def kernel(x, z, W1, b1, W2, b2):
    sort_idx = jnp.argsort(z)
    counts = jnp.bincount(z, length=_K)
    offs = jnp.concatenate(
        [jnp.zeros((1,), jnp.int32), jnp.cumsum(counts).astype(jnp.int32)])
    xs = x[sort_idx].astype(jnp.bfloat16)
    out_sorted = _grouped_mlp(offs, xs, W1, b1[:, None, :], W2, b2[:, None, :])
    return jnp.zeros((_N, 2), jnp.float32).at[sort_idx].set(out_sorted)
